# baseline (speedup 1.0000x reference)
import functools

import jax
import jax.numpy as jnp
from jax import lax
from jax.experimental import pallas as pl
from jax.experimental.pallas import tpu as pltpu
from jax.experimental.pallas import tpu_sc as plsc

NUM_CLASSES = 100000
ALPHA = 0.9
TOPRATIO = 0.01
BATCH = 1024

CLAMP_LO = -25.0
CLAMP_HI = 25.0
NB = 1024
CBLK = 1024
SC_CH = 20000
N_WORKERS = 32


def _sc_hist_body(x_hbm, out_hbm, dbuf, h16, hmerge, *, ncols, rows_per):
    wid = lax.axis_index("s") * 2 + lax.axis_index("c")
    lanes = lax.iota(jnp.int32, 16)
    laneoff = lanes * NB
    ones16 = jnp.ones((16,), jnp.float32)
    invw = jnp.float32(NB / (CLAMP_HI - CLAMP_LO))
    nchunks = ncols // SC_CH

    def do_row(ri, _):
        row = wid * rows_per + ri

        def zero_body(j, _):
            h16[pl.ds(j * 16, 16)] = jnp.zeros((16,), jnp.float32)
            return 0
        lax.fori_loop(0, NB, zero_body, 0)

        def chunk_body(ch, _):
            pltpu.sync_copy(
                x_hbm.at[pl.ds(row * ncols + ch * SC_CH, SC_CH)], dbuf)

            unroll = 25

            def inner(i, _):
                for u in range(unroll):
                    v = dbuf[pl.ds((i * unroll + u) * 16, 16)]
                    xc = jnp.minimum(jnp.maximum(v, CLAMP_LO), CLAMP_HI)
                    binf = (xc - CLAMP_LO) * invw
                    b = jnp.minimum(binf.astype(jnp.int32), NB - 1)
                    plsc.addupdate_scatter(h16, [b + laneoff], ones16)
                return 0
            lax.fori_loop(0, SC_CH // 16 // unroll, inner, 0)
            return 0
        lax.fori_loop(0, nchunks, chunk_body, 0)

        def merge_body(j, _):
            acc = h16[pl.ds(j * 16, 16)]
            for l in range(1, 16):
                acc = acc + h16[pl.ds(l * NB + j * 16, 16)]
            hmerge[pl.ds(j * 16, 16)] = acc
            return 0
        lax.fori_loop(0, NB // 16, merge_body, 0)

        pltpu.sync_copy(hmerge, out_hbm.at[pl.ds(row * NB, NB)])
        return 0
    lax.fori_loop(0, rows_per, do_row, 0)


def _sc_hist(x):
    nrows, ncols = x.shape
    rows_per = nrows // N_WORKERS
    x1d = x.reshape(nrows * ncols)
    body = functools.partial(_sc_hist_body, ncols=ncols, rows_per=rows_per)
    out = pl.kernel(
        body,
        out_type=jax.ShapeDtypeStruct((nrows * NB,), jnp.float32),
        mesh=plsc.VectorSubcoreMesh(core_axis_name="c", subcore_axis_name="s"),
        scratch_types=[
            pltpu.VMEM((SC_CH,), jnp.float32),
            pltpu.VMEM((NB * 16,), jnp.float32),
            pltpu.VMEM((NB,), jnp.float32),
        ],
        compiler_params=pltpu.CompilerParams(needs_layout_passes=False),
    )(x1d)
    return out.reshape(nrows, NB)


def _tc_body(hist_ref, x_ref, tgt_ref, out_ref, rng_ref, fin_ref, *,
             nrows, ncols, cblk, ncb, k, alpha):
    i = pl.program_id(0)
    kf = jnp.float32(k)

    @pl.when(i == 0)
    def _():
        cnt = hist_ref[...]
        r_i = lax.broadcasted_iota(jnp.int32, (NB, NB), 0)
        c_i = lax.broadcasted_iota(jnp.int32, (NB, NB), 1)
        lt = (r_i <= c_i).astype(jnp.float32)
        cum = jnp.dot(cnt, lt, preferred_element_type=jnp.float32)
        tot = cum[:, NB - 1:NB]
        jstar = jnp.sum((cum < tot - kf).astype(jnp.float32),
                        axis=1, keepdims=True)
        w = jnp.float32((CLAMP_HI - CLAMP_LO) / NB)
        lo = CLAMP_LO + jstar * w
        rng_ref[:, 0:1] = lo
        rng_ref[:, 1:2] = lo + w
        fin_ref[...] = jnp.zeros_like(fin_ref)
        out_ref[...] = jnp.zeros((1, 1), jnp.float32)

    @pl.when(i > 0)
    def _():
        cb = i - 1
        x = x_ref[...]
        tgt = tgt_ref[...]
        cols = lax.broadcasted_iota(jnp.int32, (nrows, cblk), 1) + cb * cblk
        valid = cols < ncols
        is_t = cols == tgt
        xc = jnp.where(is_t | ~valid, CLAMP_LO,
                       jnp.clip(x, CLAMP_LO, CLAMP_HI))
        lo = rng_ref[:, 0:1]
        hi = rng_ref[:, 1:2]
        sp = jnp.maximum(x, 0.0) + jnp.log1p(jnp.exp(-jnp.abs(x)))
        m = jnp.where(valid, sp, 0.0)
        above = (xc > hi).astype(jnp.float32)
        inbin = ((xc > lo) & (xc <= hi)).astype(jnp.float32)
        fin_ref[:, 0:1] += jnp.sum(m * above, axis=1, keepdims=True)
        fin_ref[:, 1:2] += jnp.sum(above, axis=1, keepdims=True)
        fin_ref[:, 2:3] += jnp.sum(m * inbin, axis=1, keepdims=True)
        fin_ref[:, 3:4] += jnp.sum(inbin, axis=1, keepdims=True)
        fin_ref[:, 4:5] += jnp.sum(
            jnp.where(is_t & valid, x - sp, 0.0), axis=1, keepdims=True)

    @pl.when(i == ncb)
    def _():
        s_above = fin_ref[:, 0:1]
        a = fin_ref[:, 1:2]
        bin_sum = fin_ref[:, 2:3]
        bin_cnt = fin_ref[:, 3:4]
        pos = fin_ref[:, 4:5]
        rem = jnp.maximum(kf - a, 0.0)
        avg = bin_sum / jnp.maximum(bin_cnt, 1.0)
        s = s_above + rem * avg
        total = (-1.0 / nrows) * jnp.sum(pos) + \
                (alpha / (nrows * kf)) * jnp.sum(s)
        out_ref[...] = total.reshape(1, 1)


def _tc_final(x, targets, hist, *, k, cblk=CBLK, alpha=ALPHA,
              interpret=False):
    nrows, ncols = x.shape
    ncb = (ncols + cblk - 1) // cblk
    tgt2 = targets.reshape(nrows, 1).astype(jnp.int32)
    body = functools.partial(_tc_body, nrows=nrows, ncols=ncols, cblk=cblk,
                             ncb=ncb, k=k, alpha=alpha)
    out = pl.pallas_call(
        body,
        grid=(ncb + 1,),
        in_specs=[
            pl.BlockSpec((nrows, NB), lambda i: (0, 0)),
            pl.BlockSpec((nrows, cblk),
                         lambda i: (0, jnp.maximum(i - 1, 0))),
            pl.BlockSpec((nrows, 1), lambda i: (0, 0)),
        ],
        out_specs=pl.BlockSpec((1, 1), lambda i: (0, 0)),
        out_shape=jax.ShapeDtypeStruct((1, 1), jnp.float32),
        scratch_shapes=[
            pltpu.VMEM((nrows, 2), jnp.float32),
            pltpu.VMEM((nrows, 8), jnp.float32),
        ],
        compiler_params=pltpu.CompilerParams(
            dimension_semantics=("arbitrary",)),
        interpret=interpret,
    )(hist, x, tgt2)
    return out[0, 0]


def kernel(inputs, targets):
    k = int(NUM_CLASSES * TOPRATIO)
    hist = _sc_hist(inputs)
    return _tc_final(inputs, targets, hist, k=k)


# speedup vs baseline: 1.0002x; 1.0002x over previous
import functools

import jax
import jax.numpy as jnp
from jax import lax
from jax.experimental import pallas as pl
from jax.experimental.pallas import tpu as pltpu
from jax.experimental.pallas import tpu_sc as plsc

NUM_CLASSES = 100000
ALPHA = 0.9
TOPRATIO = 0.01
BATCH = 1024

CLAMP_LO = -25.0
CLAMP_HI = 25.0
NB = 1024
CBLK = 1024
SC_CH = 20000
N_WORKERS = 32


def _sc_hist_body(x_hbm, out_hbm, dbuf, h16, hmerge, *, ncols, rows_per):
    wid = lax.axis_index("s") * 2 + lax.axis_index("c")
    lanes = lax.iota(jnp.int32, 16)
    ones16 = jnp.ones((16,), jnp.float32)
    invw = jnp.float32(NB / (CLAMP_HI - CLAMP_LO))
    nchunks = ncols // SC_CH

    def do_row(ri, _):
        row = wid * rows_per + ri

        def zero_body(j, _):
            h16[pl.ds(j * 16, 16)] = jnp.zeros((16,), jnp.float32)
            return 0
        lax.fori_loop(0, NB, zero_body, 0)

        def chunk_body(ch, _):
            pltpu.sync_copy(
                x_hbm.at[pl.ds(row * ncols + ch * SC_CH, SC_CH)], dbuf)

            unroll = 25

            def inner(i, _):
                for u in range(unroll):
                    v = dbuf[pl.ds((i * unroll + u) * 16, 16)]
                    xc = jnp.minimum(jnp.maximum(v, CLAMP_LO), CLAMP_HI)
                    binf = (xc - CLAMP_LO) * invw
                    b = jnp.minimum(binf.astype(jnp.int32), NB - 1)
                    plsc.addupdate_scatter(h16, [b * 16 + lanes], ones16)
                return 0
            lax.fori_loop(0, SC_CH // 16 // unroll, inner, 0)
            return 0
        lax.fori_loop(0, nchunks, chunk_body, 0)

        def merge_body(j, _):
            base = j * 256 + lanes * 16
            acc = plsc.load_gather(h16, [base])
            for i in range(1, 16):
                acc = acc + plsc.load_gather(h16, [base + i])
            hmerge[pl.ds(j * 16, 16)] = acc
            return 0
        lax.fori_loop(0, NB // 16, merge_body, 0)

        pltpu.sync_copy(hmerge, out_hbm.at[pl.ds(row * NB, NB)])
        return 0
    lax.fori_loop(0, rows_per, do_row, 0)


def _sc_hist(x):
    nrows, ncols = x.shape
    rows_per = nrows // N_WORKERS
    x1d = x.reshape(nrows * ncols)
    body = functools.partial(_sc_hist_body, ncols=ncols, rows_per=rows_per)
    out = pl.kernel(
        body,
        out_type=jax.ShapeDtypeStruct((nrows * NB,), jnp.float32),
        mesh=plsc.VectorSubcoreMesh(core_axis_name="c", subcore_axis_name="s"),
        scratch_types=[
            pltpu.VMEM((SC_CH,), jnp.float32),
            pltpu.VMEM((NB * 16,), jnp.float32),
            pltpu.VMEM((NB,), jnp.float32),
        ],
        compiler_params=pltpu.CompilerParams(needs_layout_passes=False),
    )(x1d)
    return out.reshape(nrows, NB)


def _tc_body(hist_ref, x_ref, tgt_ref, out_ref, rng_ref, fin_ref, *,
             nrows, ncols, cblk, ncb, k, alpha):
    i = pl.program_id(0)
    kf = jnp.float32(k)

    @pl.when(i == 0)
    def _():
        cnt = hist_ref[...]
        r_i = lax.broadcasted_iota(jnp.int32, (NB, NB), 0)
        c_i = lax.broadcasted_iota(jnp.int32, (NB, NB), 1)
        lt = (r_i <= c_i).astype(jnp.float32)
        cum = jnp.dot(cnt, lt, preferred_element_type=jnp.float32)
        tot = cum[:, NB - 1:NB]
        jstar = jnp.sum((cum < tot - kf).astype(jnp.float32),
                        axis=1, keepdims=True)
        w = jnp.float32((CLAMP_HI - CLAMP_LO) / NB)
        lo = CLAMP_LO + jstar * w
        rng_ref[:, 0:1] = lo
        rng_ref[:, 1:2] = lo + w
        fin_ref[...] = jnp.zeros_like(fin_ref)
        out_ref[...] = jnp.zeros((1, 1), jnp.float32)

    @pl.when(i > 0)
    def _():
        cb = i - 1
        x = x_ref[...]
        tgt = tgt_ref[...]
        cols = lax.broadcasted_iota(jnp.int32, (nrows, cblk), 1) + cb * cblk
        valid = cols < ncols
        is_t = cols == tgt
        xc = jnp.where(is_t | ~valid, CLAMP_LO,
                       jnp.clip(x, CLAMP_LO, CLAMP_HI))
        lo = rng_ref[:, 0:1]
        hi = rng_ref[:, 1:2]
        sp = jnp.maximum(x, 0.0) + jnp.log1p(jnp.exp(-jnp.abs(x)))
        m = jnp.where(valid, sp, 0.0)
        above = (xc > hi).astype(jnp.float32)
        inbin = ((xc > lo) & (xc <= hi)).astype(jnp.float32)
        fin_ref[:, 0:1] += jnp.sum(m * above, axis=1, keepdims=True)
        fin_ref[:, 1:2] += jnp.sum(above, axis=1, keepdims=True)
        fin_ref[:, 2:3] += jnp.sum(m * inbin, axis=1, keepdims=True)
        fin_ref[:, 3:4] += jnp.sum(inbin, axis=1, keepdims=True)
        fin_ref[:, 4:5] += jnp.sum(
            jnp.where(is_t & valid, x - sp, 0.0), axis=1, keepdims=True)

    @pl.when(i == ncb)
    def _():
        s_above = fin_ref[:, 0:1]
        a = fin_ref[:, 1:2]
        bin_sum = fin_ref[:, 2:3]
        bin_cnt = fin_ref[:, 3:4]
        pos = fin_ref[:, 4:5]
        rem = jnp.maximum(kf - a, 0.0)
        avg = bin_sum / jnp.maximum(bin_cnt, 1.0)
        s = s_above + rem * avg
        total = (-1.0 / nrows) * jnp.sum(pos) + \
                (alpha / (nrows * kf)) * jnp.sum(s)
        out_ref[...] = total.reshape(1, 1)


def _tc_final(x, targets, hist, *, k, cblk=CBLK, alpha=ALPHA,
              interpret=False):
    nrows, ncols = x.shape
    ncb = (ncols + cblk - 1) // cblk
    tgt2 = targets.reshape(nrows, 1).astype(jnp.int32)
    body = functools.partial(_tc_body, nrows=nrows, ncols=ncols, cblk=cblk,
                             ncb=ncb, k=k, alpha=alpha)
    out = pl.pallas_call(
        body,
        grid=(ncb + 1,),
        in_specs=[
            pl.BlockSpec((nrows, NB), lambda i: (0, 0)),
            pl.BlockSpec((nrows, cblk),
                         lambda i: (0, jnp.maximum(i - 1, 0))),
            pl.BlockSpec((nrows, 1), lambda i: (0, 0)),
        ],
        out_specs=pl.BlockSpec((1, 1), lambda i: (0, 0)),
        out_shape=jax.ShapeDtypeStruct((1, 1), jnp.float32),
        scratch_shapes=[
            pltpu.VMEM((nrows, 2), jnp.float32),
            pltpu.VMEM((nrows, 8), jnp.float32),
        ],
        compiler_params=pltpu.CompilerParams(
            dimension_semantics=("arbitrary",)),
        interpret=interpret,
    )(hist, x, tgt2)
    return out[0, 0]


def kernel(inputs, targets):
    k = int(NUM_CLASSES * TOPRATIO)
    hist = _sc_hist(inputs)
    return _tc_final(inputs, targets, hist, k=k)


# speedup vs baseline: 1.0178x; 1.0176x over previous
import functools

import jax
import jax.numpy as jnp
from jax import lax
from jax.experimental import pallas as pl
from jax.experimental.pallas import tpu as pltpu
from jax.experimental.pallas import tpu_sc as plsc

NUM_CLASSES = 100000
ALPHA = 0.9
TOPRATIO = 0.01
BATCH = 1024

CLAMP_LO = -25.0
CLAMP_HI = 25.0
NB = 1024
CBLK = 1024
SC_CH = 100000
N_WORKERS = 32


def _sc_hist_body(x_hbm, out_hbm, dbuf, h16, hmerge, *, ncols, rows_per):
    wid = lax.axis_index("s") * 2 + lax.axis_index("c")
    lanes = lax.iota(jnp.int32, 16)
    ones16 = jnp.ones((16,), jnp.float32)
    invw = jnp.float32(NB / (CLAMP_HI - CLAMP_LO))
    nchunks = ncols // SC_CH

    def do_row(ri, _):
        row = wid * rows_per + ri

        def zero_body(j, _):
            h16[pl.ds(j * 16, 16)] = jnp.zeros((16,), jnp.float32)
            return 0
        lax.fori_loop(0, NB, zero_body, 0)

        def chunk_body(ch, _):
            pltpu.sync_copy(
                x_hbm.at[pl.ds(row * ncols + ch * SC_CH, SC_CH)], dbuf)

            unroll = 25

            def inner(i, _):
                for u in range(unroll):
                    v = dbuf[pl.ds((i * unroll + u) * 16, 16)]
                    xc = jnp.minimum(jnp.maximum(v, CLAMP_LO), CLAMP_HI)
                    binf = (xc - CLAMP_LO) * invw
                    b = jnp.minimum(binf.astype(jnp.int32), NB - 1)
                    plsc.addupdate_scatter(h16, [b * 16 + lanes], ones16)
                return 0
            lax.fori_loop(0, SC_CH // 16 // unroll, inner, 0)
            return 0
        lax.fori_loop(0, nchunks, chunk_body, 0)

        def merge_body(j, _):
            base = j * 256 + lanes * 16
            acc = plsc.load_gather(h16, [base])
            for i in range(1, 16):
                acc = acc + plsc.load_gather(h16, [base + i])
            hmerge[pl.ds(j * 16, 16)] = acc
            return 0
        lax.fori_loop(0, NB // 16, merge_body, 0)

        pltpu.sync_copy(hmerge, out_hbm.at[pl.ds(row * NB, NB)])
        return 0
    lax.fori_loop(0, rows_per, do_row, 0)


def _sc_hist(x):
    nrows, ncols = x.shape
    rows_per = nrows // N_WORKERS
    x1d = x.reshape(nrows * ncols)
    body = functools.partial(_sc_hist_body, ncols=ncols, rows_per=rows_per)
    out = pl.kernel(
        body,
        out_type=jax.ShapeDtypeStruct((nrows * NB,), jnp.float32),
        mesh=plsc.VectorSubcoreMesh(core_axis_name="c", subcore_axis_name="s"),
        scratch_types=[
            pltpu.VMEM((SC_CH,), jnp.float32),
            pltpu.VMEM((NB * 16,), jnp.float32),
            pltpu.VMEM((NB,), jnp.float32),
        ],
        compiler_params=pltpu.CompilerParams(needs_layout_passes=False),
    )(x1d)
    return out.reshape(nrows, NB)


def _tc_body(hist_ref, x_ref, tgt_ref, out_ref, rng_ref, fin_ref, *,
             nrows, ncols, cblk, ncb, k, alpha):
    i = pl.program_id(0)
    kf = jnp.float32(k)

    @pl.when(i == 0)
    def _():
        cnt = hist_ref[...]
        r_i = lax.broadcasted_iota(jnp.int32, (NB, NB), 0)
        c_i = lax.broadcasted_iota(jnp.int32, (NB, NB), 1)
        lt = (r_i <= c_i).astype(jnp.float32)
        cum = jnp.dot(cnt, lt, preferred_element_type=jnp.float32)
        tot = cum[:, NB - 1:NB]
        jstar = jnp.sum((cum < tot - kf).astype(jnp.float32),
                        axis=1, keepdims=True)
        w = jnp.float32((CLAMP_HI - CLAMP_LO) / NB)
        lo = CLAMP_LO + jstar * w
        rng_ref[:, 0:1] = lo
        rng_ref[:, 1:2] = lo + w
        fin_ref[...] = jnp.zeros_like(fin_ref)
        out_ref[...] = jnp.zeros((1, 1), jnp.float32)

    @pl.when(i > 0)
    def _():
        cb = i - 1
        x = x_ref[...]
        tgt = tgt_ref[...]
        cols = lax.broadcasted_iota(jnp.int32, (nrows, cblk), 1) + cb * cblk
        valid = cols < ncols
        is_t = cols == tgt
        xc = jnp.where(is_t | ~valid, CLAMP_LO,
                       jnp.clip(x, CLAMP_LO, CLAMP_HI))
        lo = rng_ref[:, 0:1]
        hi = rng_ref[:, 1:2]
        sp = jnp.maximum(x, 0.0) + jnp.log1p(jnp.exp(-jnp.abs(x)))
        m = jnp.where(valid, sp, 0.0)
        above = (xc > hi).astype(jnp.float32)
        inbin = ((xc > lo) & (xc <= hi)).astype(jnp.float32)
        fin_ref[:, 0:1] += jnp.sum(m * above, axis=1, keepdims=True)
        fin_ref[:, 1:2] += jnp.sum(above, axis=1, keepdims=True)
        fin_ref[:, 2:3] += jnp.sum(m * inbin, axis=1, keepdims=True)
        fin_ref[:, 3:4] += jnp.sum(inbin, axis=1, keepdims=True)
        fin_ref[:, 4:5] += jnp.sum(
            jnp.where(is_t & valid, x - sp, 0.0), axis=1, keepdims=True)

    @pl.when(i == ncb)
    def _():
        s_above = fin_ref[:, 0:1]
        a = fin_ref[:, 1:2]
        bin_sum = fin_ref[:, 2:3]
        bin_cnt = fin_ref[:, 3:4]
        pos = fin_ref[:, 4:5]
        rem = jnp.maximum(kf - a, 0.0)
        avg = bin_sum / jnp.maximum(bin_cnt, 1.0)
        s = s_above + rem * avg
        total = (-1.0 / nrows) * jnp.sum(pos) + \
                (alpha / (nrows * kf)) * jnp.sum(s)
        out_ref[...] = total.reshape(1, 1)


def _tc_final(x, targets, hist, *, k, cblk=CBLK, alpha=ALPHA,
              interpret=False):
    nrows, ncols = x.shape
    ncb = (ncols + cblk - 1) // cblk
    tgt2 = targets.reshape(nrows, 1).astype(jnp.int32)
    body = functools.partial(_tc_body, nrows=nrows, ncols=ncols, cblk=cblk,
                             ncb=ncb, k=k, alpha=alpha)
    out = pl.pallas_call(
        body,
        grid=(ncb + 1,),
        in_specs=[
            pl.BlockSpec((nrows, NB), lambda i: (0, 0)),
            pl.BlockSpec((nrows, cblk),
                         lambda i: (0, jnp.maximum(i - 1, 0))),
            pl.BlockSpec((nrows, 1), lambda i: (0, 0)),
        ],
        out_specs=pl.BlockSpec((1, 1), lambda i: (0, 0)),
        out_shape=jax.ShapeDtypeStruct((1, 1), jnp.float32),
        scratch_shapes=[
            pltpu.VMEM((nrows, 2), jnp.float32),
            pltpu.VMEM((nrows, 8), jnp.float32),
        ],
        compiler_params=pltpu.CompilerParams(
            dimension_semantics=("arbitrary",)),
        interpret=interpret,
    )(hist, x, tgt2)
    return out[0, 0]


def kernel(inputs, targets):
    k = int(NUM_CLASSES * TOPRATIO)
    hist = _sc_hist(inputs)
    return _tc_final(inputs, targets, hist, k=k)
